# Initial kernel scaffold; baseline (speedup 1.0000x reference)
import jax
import jax.numpy as jnp
from jax.experimental import pallas as pl


def kernel(x, idx, num_users, weight, name_emb):
    raise NotImplementedError("write your pallas kernel here")


# speedup vs baseline: 3.9056x; 3.9056x over previous
import functools

import jax
import jax.numpy as jnp
from jax import lax
from jax.experimental import pallas as pl
from jax.experimental.pallas import tpu as pltpu
from jax.experimental.pallas import tpu_sc as plsc

_NC = 2
_NS = 16
_NW = _NC * _NS
_L = 16

_B = 16384
_D = 64
_BPW = _B // _NW
_CH = 128
_NCH = _BPW // _CH
_FROWS = 112
_ZROW = 104


def _fused_body(wl_ref, wc_ref, o_ref):
    k4 = lax.broadcasted_iota(jnp.int32, (_FROWS, 4), 0)
    j4 = lax.broadcasted_iota(jnp.int32, (_FROWS, 4), 1)
    a = ((k4 // 26) == j4).astype(jnp.float32)
    k26 = lax.broadcasted_iota(jnp.int32, (_FROWS, 26), 0)
    j26 = lax.broadcasted_iota(jnp.int32, (_FROWS, 26), 1)
    b = (((k26 % 26) == j26) & (k26 < 104)).astype(jnp.float32)
    o_ref[...] = (
        jnp.dot(a, wl_ref[...], preferred_element_type=jnp.float32)
        + jnp.dot(b, wc_ref[...], preferred_element_type=jnp.float32)
    )


def _build_fused(wl, wc):
    return pl.pallas_call(
        _fused_body,
        out_shape=jax.ShapeDtypeStruct((_FROWS, _D), jnp.float32),
    )(wl, wc)


def _sc_body(xf_hbm, idx_hbm, nu_hbm, w_hbm, nm_hbm, fu_hbm, out_hbm,
             idx_v, nu_v, e1_v, e2_v, lv_v, in_v, g1_v, g2_v, g3_v,
             w_v, n_v, f_v, sem, sem2):
    wid = lax.axis_index("s") * _NC + lax.axis_index("c")
    base = wid * _BPW

    pltpu.sync_copy(idx_hbm.at[pl.ds(base, _BPW)], idx_v)
    pltpu.sync_copy(nu_hbm, nu_v)

    nu = nu_v[...]
    for i in range(_BPW // _L):
        sl = pl.ds(i * _L, _L)
        idxc = idx_v[sl]
        item = idxc >= nu
        g1_v[sl] = jnp.where(item, idxc + 30, idxc)
        g2_v[sl] = jnp.where(item, idxc + 30, 0)
        e1_v[sl] = idxc * 3 + 1
        e2_v[sl] = idxc * 3 + 2

    descs = []
    xdescs = []
    for j in range(_NCH):
        sl = pl.ds(j * _CH, _CH)
        descs.append(pltpu.async_copy(w_hbm.at[g1_v.at[sl]], w_v.at[sl], sem))
        descs.append(pltpu.async_copy(nm_hbm.at[g2_v.at[sl]], n_v.at[sl], sem))
        xdescs.append(pltpu.async_copy(xf_hbm.at[e1_v.at[sl]], lv_v.at[sl], sem2))
        xdescs.append(pltpu.async_copy(xf_hbm.at[e2_v.at[sl]], in_v.at[sl], sem2))
    for d in xdescs:
        d.wait()

    for i in range(_BPW // _L):
        sl = pl.ds(i * _L, _L)
        item = idx_v[sl] >= nu
        g3_v[sl] = jnp.where(item, _ZROW, lv_v[sl] * 26 + in_v[sl])

    for j in range(_NCH):
        sl = pl.ds(j * _CH, _CH)
        descs.append(pltpu.async_copy(fu_hbm.at[g3_v.at[sl]], f_v.at[sl], sem))
    for d in descs:
        d.wait()

    @plsc.parallel_loop(0, _BPW)
    def _add(b):
        for j in range(_D // _L):
            sl = pl.ds(j * _L, _L)
            w_v[b, sl] = w_v[b, sl] + n_v[b, sl] + f_v[b, sl]

    pltpu.sync_copy(w_v, out_hbm.at[pl.ds(base, _BPW)])


@functools.partial(jax.jit, static_argnames=())
def _sc_lookup(x, idx, nu_vec, weight, name_emb, fused):
    mesh = plsc.VectorSubcoreMesh(core_axis_name="c", subcore_axis_name="s")
    return pl.kernel(
        _sc_body,
        out_type=jax.ShapeDtypeStruct((_B, _D), jnp.float32),
        mesh=mesh,
        compiler_params=pltpu.CompilerParams(use_tc_tiling_on_sc=False),
        scratch_types=[
            pltpu.VMEM((_BPW,), jnp.int32),
            pltpu.VMEM((_L,), jnp.int32),
            pltpu.VMEM((_BPW,), jnp.int32),
            pltpu.VMEM((_BPW,), jnp.int32),
            pltpu.VMEM((_BPW,), jnp.int32),
            pltpu.VMEM((_BPW,), jnp.int32),
            pltpu.VMEM((_BPW,), jnp.int32),
            pltpu.VMEM((_BPW,), jnp.int32),
            pltpu.VMEM((_BPW,), jnp.int32),
            pltpu.VMEM((_BPW, _D), jnp.float32),
            pltpu.VMEM((_BPW, _D), jnp.float32),
            pltpu.VMEM((_BPW, _D), jnp.float32),
            pltpu.SemaphoreType.DMA,
            pltpu.SemaphoreType.DMA,
        ],
    )(x, idx, nu_vec, weight, name_emb, fused)


def kernel(x, idx, num_users, weight, name_emb):
    x = x.astype(jnp.int32).reshape(-1)
    idx = idx.astype(jnp.int32)
    nu = jnp.asarray(num_users, jnp.int32)
    wl = lax.dynamic_slice_in_dim(weight, nu, 4, axis=0)
    wc = lax.dynamic_slice_in_dim(weight, nu + 4, 26, axis=0)
    fused = _build_fused(wl, wc)
    nu_vec = jnp.full((_L,), nu, jnp.int32)
    return _sc_lookup(x, idx, nu_vec, weight, name_emb, fused)


# speedup vs baseline: 3.9079x; 1.0006x over previous
import functools

import jax
import jax.numpy as jnp
from jax import lax
from jax.experimental import pallas as pl
from jax.experimental.pallas import tpu as pltpu
from jax.experimental.pallas import tpu_sc as plsc

_NC = 2
_NS = 16
_NW = _NC * _NS
_L = 16

_B = 16384
_D = 64
_BPW = _B // _NW
_CH = 128
_NCH = _BPW // _CH
_FROWS = 112
_ZROW = 104


def _fused_body(wl_ref, wc_ref, o_ref):
    k4 = lax.broadcasted_iota(jnp.int32, (_FROWS, 4), 0)
    j4 = lax.broadcasted_iota(jnp.int32, (_FROWS, 4), 1)
    a = ((k4 // 26) == j4).astype(jnp.float32)
    k26 = lax.broadcasted_iota(jnp.int32, (_FROWS, 26), 0)
    j26 = lax.broadcasted_iota(jnp.int32, (_FROWS, 26), 1)
    b = (((k26 % 26) == j26) & (k26 < 104)).astype(jnp.float32)
    o_ref[...] = (
        jnp.dot(a, wl_ref[...], preferred_element_type=jnp.float32)
        + jnp.dot(b, wc_ref[...], preferred_element_type=jnp.float32)
    )


def _build_fused(wl, wc):
    return pl.pallas_call(
        _fused_body,
        out_shape=jax.ShapeDtypeStruct((_FROWS, _D), jnp.float32),
    )(wl, wc)


def _sc_body(xf_hbm, idx_hbm, nu_hbm, w_hbm, nm_hbm, fu_hbm, out_hbm,
             idx_v, nu_v, e1_v, e2_v, lv_v, in_v, g1_v, g2_v, g3_v,
             w_v, n_v, f_v, sem, sem2):
    wid = lax.axis_index("s") * _NC + lax.axis_index("c")
    base = wid * _BPW

    with jax.named_scope("stage_idx"):
        pltpu.sync_copy(idx_hbm.at[pl.ds(base, _BPW)], idx_v)
        pltpu.sync_copy(nu_hbm, nu_v)

    nu = nu_v[...]
    with jax.named_scope("pass1"):
        for i in range(_BPW // _L):
            sl = pl.ds(i * _L, _L)
            idxc = idx_v[sl]
            item = idxc >= nu
            g1_v[sl] = jnp.where(item, idxc + 30, idxc)
            g2_v[sl] = jnp.where(item, idxc + 30, 0)
            e1_v[sl] = idxc * 3 + 1
            e2_v[sl] = idxc * 3 + 2

    with jax.named_scope("fire_gathers"):
        descs = []
        xdescs = []
        for j in range(_NCH):
            sl = pl.ds(j * _CH, _CH)
            descs.append(pltpu.async_copy(w_hbm.at[g1_v.at[sl]], w_v.at[sl], sem))
            descs.append(pltpu.async_copy(nm_hbm.at[g2_v.at[sl]], n_v.at[sl], sem))
            xdescs.append(pltpu.async_copy(xf_hbm.at[e1_v.at[sl]], lv_v.at[sl], sem2))
            xdescs.append(pltpu.async_copy(xf_hbm.at[e2_v.at[sl]], in_v.at[sl], sem2))
    with jax.named_scope("wait_x"):
        for d in xdescs:
            d.wait()

    with jax.named_scope("pass2"):
        for i in range(_BPW // _L):
            sl = pl.ds(i * _L, _L)
            item = idx_v[sl] >= nu
            g3_v[sl] = jnp.where(item, _ZROW, lv_v[sl] * 26 + in_v[sl])

        for j in range(_NCH):
            sl = pl.ds(j * _CH, _CH)
            descs.append(pltpu.async_copy(fu_hbm.at[g3_v.at[sl]], f_v.at[sl], sem))
    with jax.named_scope("wait_gathers"):
        for d in descs:
            d.wait()

    with jax.named_scope("addloop"):
        @plsc.parallel_loop(0, _BPW)
        def _add(b):
            for j in range(_D // _L):
                sl = pl.ds(j * _L, _L)
                w_v[b, sl] = w_v[b, sl] + n_v[b, sl] + f_v[b, sl]

    with jax.named_scope("writeback"):
        pltpu.sync_copy(w_v, out_hbm.at[pl.ds(base, _BPW)])


@functools.partial(jax.jit, static_argnames=())
def _sc_lookup(x, idx, nu_vec, weight, name_emb, fused):
    mesh = plsc.VectorSubcoreMesh(core_axis_name="c", subcore_axis_name="s")
    return pl.kernel(
        _sc_body,
        out_type=jax.ShapeDtypeStruct((_B, _D), jnp.float32),
        mesh=mesh,
        compiler_params=pltpu.CompilerParams(use_tc_tiling_on_sc=False),
        scratch_types=[
            pltpu.VMEM((_BPW,), jnp.int32),
            pltpu.VMEM((_L,), jnp.int32),
            pltpu.VMEM((_BPW,), jnp.int32),
            pltpu.VMEM((_BPW,), jnp.int32),
            pltpu.VMEM((_BPW,), jnp.int32),
            pltpu.VMEM((_BPW,), jnp.int32),
            pltpu.VMEM((_BPW,), jnp.int32),
            pltpu.VMEM((_BPW,), jnp.int32),
            pltpu.VMEM((_BPW,), jnp.int32),
            pltpu.VMEM((_BPW, _D), jnp.float32),
            pltpu.VMEM((_BPW, _D), jnp.float32),
            pltpu.VMEM((_BPW, _D), jnp.float32),
            pltpu.SemaphoreType.DMA,
            pltpu.SemaphoreType.DMA,
        ],
    )(x, idx, nu_vec, weight, name_emb, fused)


def kernel(x, idx, num_users, weight, name_emb):
    x = x.astype(jnp.int32).reshape(-1)
    idx = idx.astype(jnp.int32)
    nu = jnp.asarray(num_users, jnp.int32)
    wl = lax.dynamic_slice_in_dim(weight, nu, 4, axis=0)
    wc = lax.dynamic_slice_in_dim(weight, nu + 4, 26, axis=0)
    fused = _build_fused(wl, wc)
    nu_vec = jnp.full((_L,), nu, jnp.int32)
    return _sc_lookup(x, idx, nu_vec, weight, name_emb, fused)


# speedup vs baseline: 6.0733x; 1.5541x over previous
import functools

import jax
import jax.numpy as jnp
from jax import lax
from jax.experimental import pallas as pl
from jax.experimental.pallas import tpu as pltpu
from jax.experimental.pallas import tpu_sc as plsc

_NC = 2
_NS = 16
_NW = _NC * _NS
_L = 16

_B = 16384
_D = 64
_CW = 144
_BPW = _B // _NW
_CH = 128
_NCH = _BPW // _CH
_FROWS = 112
_ZROW = 104


def _fused_body(wl_ref, wc_ref, n0_ref, o_ref):
    k4 = lax.broadcasted_iota(jnp.int32, (_FROWS, 4), 0)
    j4 = lax.broadcasted_iota(jnp.int32, (_FROWS, 4), 1)
    a = ((k4 // 26) == j4).astype(jnp.float32)
    k26 = lax.broadcasted_iota(jnp.int32, (_FROWS, 26), 0)
    j26 = lax.broadcasted_iota(jnp.int32, (_FROWS, 26), 1)
    b = (((k26 % 26) == j26) & (k26 < 104)).astype(jnp.float32)
    live = (lax.broadcasted_iota(jnp.int32, (_FROWS, 1), 0) < 104)
    o_ref[...] = (
        jnp.dot(a, wl_ref[...], preferred_element_type=jnp.float32,
                precision=lax.Precision.HIGHEST)
        + jnp.dot(b, wc_ref[...], preferred_element_type=jnp.float32,
                  precision=lax.Precision.HIGHEST)
        + jnp.where(live, n0_ref[...], 0.0)
    )


def _build_fused(wl, wc, n0):
    return pl.pallas_call(
        _fused_body,
        out_shape=jax.ShapeDtypeStruct((_FROWS, _D), jnp.float32),
    )(wl, wc, n0)


def _sc_body(c_hbm, idx_hbm, nu_hbm, fu_hbm, out_hbm,
             idx_v, nu_v, g1_v, nm_v, c_v, fu_v, out_v, sem):
    wid = lax.axis_index("s") * _NC + lax.axis_index("c")
    base = wid * _BPW

    with jax.named_scope("stage"):
        pltpu.sync_copy(idx_hbm.at[pl.ds(base, _BPW)], idx_v)
        pltpu.sync_copy(nu_hbm, nu_v)
        fdesc = pltpu.async_copy(fu_hbm, fu_v, sem)

    nu = nu_v[...]
    with jax.named_scope("pass1"):
        for i in range(_BPW // _L):
            sl = pl.ds(i * _L, _L)
            idxc = idx_v[sl]
            item = idxc >= nu
            g1_v[sl] = jnp.where(item, idxc + 30, idxc)
            nm_v[sl] = jnp.where(item, 1.0, 0.0)

    with jax.named_scope("gather"):
        descs = []
        for j in range(_NCH):
            sl = pl.ds(j * _CH, _CH)
            descs.append(pltpu.async_copy(c_hbm.at[g1_v.at[sl]], c_v.at[sl], sem))
        fdesc.wait()
        for d in descs:
            d.wait()

    with jax.named_scope("combine"):
        @plsc.parallel_loop(0, _BPW // _L)
        def _combine(ci):
            nm16 = nm_v[pl.ds(ci * _L, _L)]
            for k in range(_L):
                bb = ci * _L + k
                tv = c_v[bb, pl.ds(2 * _D, _L)]
                nm_s = nm16[k]
                lvl = tv[0].astype(jnp.int32)
                ins = tv[1].astype(jnp.int32)
                frow = jnp.where(nm_s > 0.5, _ZROW, lvl * 26 + ins)
                for j in range(_D // _L):
                    sl = pl.ds(j * _L, _L)
                    out_v[bb, sl] = (
                        c_v[bb, sl]
                        + nm_s * c_v[bb, pl.ds(_D + j * _L, _L)]
                        + fu_v[frow, sl]
                    )

    with jax.named_scope("writeback"):
        pltpu.sync_copy(out_v, out_hbm.at[pl.ds(base, _BPW)])


@jax.jit
def _sc_lookup(ctab, idx, nu_vec, fused):
    mesh = plsc.VectorSubcoreMesh(core_axis_name="c", subcore_axis_name="s")
    return pl.kernel(
        _sc_body,
        out_type=jax.ShapeDtypeStruct((_B, _D), jnp.float32),
        mesh=mesh,
        compiler_params=pltpu.CompilerParams(use_tc_tiling_on_sc=False),
        scratch_types=[
            pltpu.VMEM((_BPW,), jnp.int32),
            pltpu.VMEM((_L,), jnp.int32),
            pltpu.VMEM((_BPW,), jnp.int32),
            pltpu.VMEM((_BPW,), jnp.float32),
            pltpu.VMEM((_BPW, _CW), jnp.float32),
            pltpu.VMEM((_FROWS, _D), jnp.float32),
            pltpu.VMEM((_BPW, _D), jnp.float32),
            pltpu.SemaphoreType.DMA,
        ],
    )(ctab, idx, nu_vec, fused)


def kernel(x, idx, num_users, weight, name_emb):
    x = x.astype(jnp.int32)
    idx = idx.astype(jnp.int32)
    nu = jnp.asarray(num_users, jnp.int32)
    wl = lax.dynamic_slice_in_dim(weight, nu, 4, axis=0)
    wc = lax.dynamic_slice_in_dim(weight, nu + 4, 26, axis=0)
    fused = _build_fused(wl, wc, name_emb[0:1])
    nrows = weight.shape[0]
    xc = jnp.zeros((nrows, 2), jnp.float32)
    xc = lax.dynamic_update_slice(xc, x[:, 1:3].astype(jnp.float32), (0, 0))
    pad = jnp.zeros((nrows, _CW - 2 * _D - 2), jnp.float32)
    ctab = jnp.concatenate([weight, name_emb, xc, pad], axis=1)
    nu_vec = jnp.full((_L,), nu, jnp.int32)
    return _sc_lookup(ctab, idx, nu_vec, fused)
